# baseline (speedup 1.0000x reference)
import functools

import jax
import jax.numpy as jnp
from jax import lax
from jax.experimental import pallas as pl
from jax.experimental.pallas import tpu as pltpu
from jax.experimental.pallas import tpu_sc as plsc

_INFO = plsc.get_sparse_core_info()
_NC = _INFO.num_cores
_NS = _INFO.num_subcores
_NW = _NC * _NS
_L = _INFO.num_lanes

_ROWS = 512
_NBUF = 2


def _make_gather(n_rows, vocab, dim):
    assert n_rows % _NW == 0
    b_per_w = n_rows // _NW
    assert b_per_w % (_ROWS * _NBUF) == 0
    n_it = b_per_w // _ROWS
    n_groups = n_it // _NBUF
    n_blocks = _ROWS // _L

    @functools.partial(
        pl.kernel,
        mesh=plsc.VectorSubcoreMesh(core_axis_name="c", subcore_axis_name="s"),
        out_type=jax.ShapeDtypeStruct((n_rows, dim), jnp.float32),
        scratch_types=[
            pltpu.VMEM((b_per_w,), jnp.int32),
            pltpu.VMEM((vocab, dim), jnp.float32),
            pltpu.VMEM((_NBUF, _ROWS, dim), jnp.float32),
        ]
        + [pltpu.SemaphoreType.DMA] * _NBUF,
        compiler_params=pltpu.CompilerParams(
            use_tc_tiling_on_sc=False, needs_layout_passes=False
        ),
    )
    def gather_kernel(table_hbm, idx_hbm, out_hbm, idx_v, table_v, rows, *ssem):
        wid = lax.axis_index("s") * _NC + lax.axis_index("c")
        base = pl.multiple_of(wid * b_per_w, _ROWS)
        pltpu.sync_copy(idx_hbm.at[pl.ds(base, b_per_w)], idx_v)
        pltpu.sync_copy(table_hbm, table_v)

        lane = lax.iota(jnp.int32, _L)

        def fill(i, b):
            rows_b = rows.at[b]

            def blk_body(blk, carry):
                rr0 = blk * _L
                idxv = idx_v[pl.ds(pl.multiple_of(i * _ROWS, _L) + rr0, _L)]
                rposv = jnp.full((_L,), rr0, jnp.int32) + lane
                for c in range(dim):
                    colv = jnp.full((_L,), c, jnp.int32)
                    vals = plsc.load_gather(table_v, [idxv, colv])
                    plsc.store_scatter(rows_b, [rposv, colv], vals)
                return carry

            lax.fori_loop(0, n_blocks, blk_body, 0)

        def fire_store(i, b):
            off = pl.multiple_of(i * _ROWS, _ROWS)
            pltpu.async_copy(rows.at[b], out_hbm.at[pl.ds(base + off, _ROWS)], ssem[b])

        def wait_store(b):
            pltpu.make_async_copy(
                out_hbm.at[pl.ds(0, _ROWS)], rows.at[b], ssem[b]
            ).wait()

        def body(g, carry):
            for b in range(_NBUF):
                i = g * _NBUF + b

                @pl.when(g > 0)
                def _():
                    wait_store(b)

                fill(i, b)
                fire_store(i, b)
            return carry

        lax.fori_loop(0, n_groups, body, 0)
        for b in range(_NBUF):
            wait_store(b)

    return gather_kernel


def kernel(indices, table):
    batch, hist = indices.shape
    vocab, dim = table.shape
    n_rows = batch * hist
    idx_flat = indices.reshape(n_rows)
    out = _make_gather(n_rows, vocab, dim)(table, idx_flat)
    return out.reshape(batch, hist, dim)


# speedup vs baseline: 1.2210x; 1.2210x over previous
import functools

import jax
import jax.numpy as jnp
from jax import lax
from jax.experimental import pallas as pl
from jax.experimental.pallas import tpu as pltpu
from jax.experimental.pallas import tpu_sc as plsc

_INFO = plsc.get_sparse_core_info()
_NC = _INFO.num_cores
_NS = _INFO.num_subcores
_NW = _NC * _NS
_L = _INFO.num_lanes

_ROWS = 512
_NBUF = 2


def _make_gather(n_rows, vocab, dim):
    assert n_rows % _NW == 0
    b_per_w = n_rows // _NW
    assert b_per_w % (_ROWS * _NBUF) == 0
    n_it = b_per_w // _ROWS
    n_groups = n_it // _NBUF
    n_blocks = _ROWS // _L

    @functools.partial(
        pl.kernel,
        mesh=plsc.VectorSubcoreMesh(core_axis_name="c", subcore_axis_name="s"),
        out_type=jax.ShapeDtypeStruct((n_rows, dim), jnp.float32),
        scratch_types=[
            pltpu.VMEM((b_per_w,), jnp.int32),
            pltpu.VMEM((vocab, dim), jnp.float32),
            pltpu.VMEM((_NBUF, _ROWS, dim), jnp.float32),
        ]
        + [pltpu.SemaphoreType.DMA] * _NBUF,
        compiler_params=pltpu.CompilerParams(
            use_tc_tiling_on_sc=False, needs_layout_passes=False
        ),
    )
    def gather_kernel(table_hbm, idx_hbm, out_hbm, idx_v, table_v, rows, *ssem):
        wid = lax.axis_index("s") * _NC + lax.axis_index("c")
        base = pl.multiple_of(wid * b_per_w, _ROWS)
        pltpu.sync_copy(idx_hbm.at[pl.ds(base, b_per_w)], idx_v)
        pltpu.sync_copy(table_hbm, table_v)

        lane = lax.iota(jnp.int32, _L)

        def fill(i, b):
            rows_b = rows.at[b]

            @plsc.parallel_loop(0, n_blocks, unroll=2)
            def blk_body(blk):
                rr0 = blk * _L
                idxv = idx_v[pl.ds(pl.multiple_of(i * _ROWS, _L) + rr0, _L)]
                rposv = jnp.full((_L,), rr0, jnp.int32) + lane
                for c in range(dim):
                    colv = jnp.full((_L,), c, jnp.int32)
                    vals = plsc.load_gather(table_v, [idxv, colv])
                    plsc.store_scatter(rows_b, [rposv, colv], vals)

        def fire_store(i, b):
            off = pl.multiple_of(i * _ROWS, _ROWS)
            pltpu.async_copy(rows.at[b], out_hbm.at[pl.ds(base + off, _ROWS)], ssem[b])

        def wait_store(b):
            pltpu.make_async_copy(
                out_hbm.at[pl.ds(0, _ROWS)], rows.at[b], ssem[b]
            ).wait()

        def body(g, carry):
            for b in range(_NBUF):
                i = g * _NBUF + b

                @pl.when(g > 0)
                def _():
                    wait_store(b)

                fill(i, b)
                fire_store(i, b)
            return carry

        lax.fori_loop(0, n_groups, body, 0)
        for b in range(_NBUF):
            wait_store(b)

    return gather_kernel


def kernel(indices, table):
    batch, hist = indices.shape
    vocab, dim = table.shape
    n_rows = batch * hist
    idx_flat = indices.reshape(n_rows)
    out = _make_gather(n_rows, vocab, dim)(table, idx_flat)
    return out.reshape(batch, hist, dim)


# speedup vs baseline: 3.6373x; 2.9789x over previous
import functools

import jax
import jax.numpy as jnp
from jax import lax
from jax.experimental import pallas as pl
from jax.experimental.pallas import tpu as pltpu
from jax.experimental.pallas import tpu_sc as plsc

_INFO = plsc.get_sparse_core_info()
_NC = _INFO.num_cores
_NS = _INFO.num_subcores
_NW = _NC * _NS
_L = _INFO.num_lanes

_ROWS = 512
_NBUF = 2


def _make_gather(n_rows, vocab, dim):
    assert n_rows % _NW == 0
    b_per_w = n_rows // _NW
    assert b_per_w % (_ROWS * _NBUF) == 0
    n_it = b_per_w // _ROWS
    n_groups = n_it // _NBUF
    n_blocks = _ROWS // _L

    @functools.partial(
        pl.kernel,
        mesh=plsc.VectorSubcoreMesh(core_axis_name="c", subcore_axis_name="s"),
        out_type=jax.ShapeDtypeStruct((n_rows, dim), jnp.float32),
        scratch_types=[
            pltpu.VMEM((b_per_w,), jnp.int32),
            pltpu.VMEM((vocab, dim + 1), jnp.float32),
            pltpu.VMEM((_NBUF, _ROWS, dim), jnp.float32),
        ]
        + [pltpu.SemaphoreType.DMA] * _NBUF,
        compiler_params=pltpu.CompilerParams(
            use_tc_tiling_on_sc=False, needs_layout_passes=False
        ),
    )
    def gather_kernel(table_hbm, idx_hbm, out_hbm, idx_v, table_v, rows, *ssem):
        wid = lax.axis_index("s") * _NC + lax.axis_index("c")
        base = pl.multiple_of(wid * b_per_w, _ROWS)
        pltpu.sync_copy(idx_hbm.at[pl.ds(base, b_per_w)], idx_v)
        pltpu.sync_copy(table_hbm, table_v.at[:, pl.ds(0, dim)])

        lane = lax.iota(jnp.int32, _L)
        gdn = lax.GatherDimensionNumbers(
            offset_dims=(), collapsed_slice_dims=(0,), start_index_map=(0,)
        )

        def bcast_lane(vec, r):
            return lax.gather(
                vec,
                jnp.full((_L, 1), r, jnp.int32),
                gdn,
                (1,),
                mode=lax.GatherScatterMode.PROMISE_IN_BOUNDS,
            )

        def fill(i, b):
            rows_b = rows.at[b]

            @plsc.parallel_loop(0, n_blocks, unroll=2)
            def blk_body(blk):
                rr0 = blk * _L
                idxv = idx_v[pl.ds(pl.multiple_of(i * _ROWS, _L) + rr0, _L)]
                for r in range(_L):
                    rowv = bcast_lane(idxv, r)
                    dst = rows_b.at[rr0 + r]
                    for k in range(dim // _L):
                        colv = lane + (k * _L)
                        vals = plsc.load_gather(table_v, [rowv, colv])
                        dst[pl.ds(k * _L, _L)] = vals

        def fire_store(i, b):
            off = pl.multiple_of(i * _ROWS, _ROWS)
            pltpu.async_copy(rows.at[b], out_hbm.at[pl.ds(base + off, _ROWS)], ssem[b])

        def wait_store(b):
            pltpu.make_async_copy(
                out_hbm.at[pl.ds(0, _ROWS)], rows.at[b], ssem[b]
            ).wait()

        def body(g, carry):
            for b in range(_NBUF):
                i = g * _NBUF + b

                @pl.when(g > 0)
                def _():
                    wait_store(b)

                fill(i, b)
                fire_store(i, b)
            return carry

        lax.fori_loop(0, n_groups, body, 0)
        for b in range(_NBUF):
            wait_store(b)

    return gather_kernel


def kernel(indices, table):
    batch, hist = indices.shape
    vocab, dim = table.shape
    n_rows = batch * hist
    idx_flat = indices.reshape(n_rows)
    out = _make_gather(n_rows, vocab, dim)(table, idx_flat)
    return out.reshape(batch, hist, dim)


# speedup vs baseline: 5.7022x; 1.5677x over previous
import functools

import jax
import jax.numpy as jnp
from jax import lax
from jax.experimental import pallas as pl
from jax.experimental.pallas import tpu as pltpu
from jax.experimental.pallas import tpu_sc as plsc

_INFO = plsc.get_sparse_core_info()
_NC = _INFO.num_cores
_NS = _INFO.num_subcores
_NW = _NC * _NS
_L = _INFO.num_lanes


def _make_gather(batch, hist, vocab, dim):
    assert batch % (_NW * 128) == 0
    b_per_w = batch // _NW
    n_bblk = b_per_w // _L
    stride = dim + 1

    @functools.partial(
        pl.kernel,
        mesh=plsc.VectorSubcoreMesh(core_axis_name="c", subcore_axis_name="s"),
        out_type=jax.ShapeDtypeStruct((hist, dim, batch), jnp.float32),
        scratch_types=[
            pltpu.VMEM((b_per_w * hist,), jnp.int32),
            pltpu.VMEM((vocab * dim,), jnp.float32),
            pltpu.VMEM((vocab * stride,), jnp.float32),
            pltpu.VMEM((1, dim, b_per_w), jnp.float32),
            pltpu.VMEM((1, dim, b_per_w), jnp.float32),
            pltpu.SemaphoreType.DMA,
            pltpu.SemaphoreType.DMA,
        ],
        compiler_params=pltpu.CompilerParams(
            use_tc_tiling_on_sc=True, needs_layout_passes=False
        ),
    )
    def gather_kernel(
        table_hbm, idx_hbm, out_hbm, idx_v, stage_v, table_v, pan0, pan1, sem0, sem1
    ):
        panels = (pan0, pan1)
        ssem = (sem0, sem1)
        wid = lax.axis_index("s") * _NC + lax.axis_index("c")
        base = pl.multiple_of(wid * (b_per_w * hist), 8)
        pltpu.sync_copy(idx_hbm.at[pl.ds(base, b_per_w * hist)], idx_v)
        pltpu.sync_copy(table_hbm, stage_v)
        for r in range(vocab):
            for k in range(dim // _L):
                table_v[pl.ds(r * stride + k * _L, _L)] = stage_v[
                    pl.ds(r * dim + k * _L, _L)
                ]

        lane = lax.iota(jnp.int32, _L)
        bcol = pl.multiple_of(wid * b_per_w, 128)

        zerov = jnp.zeros((_L,), jnp.int32)

        def fill(h, pan):
            @plsc.parallel_loop(0, n_bblk, unroll=2)
            def blk_body(blk):
                b0 = blk * _L
                posv = jnp.full((_L,), b0 * hist + h, jnp.int32) + lane * hist
                idxv = plsc.load_gather(idx_v, [posv])
                rowbase = idxv * stride
                bv = jnp.full((_L,), b0, jnp.int32) + lane
                for d in range(dim):
                    vals = plsc.load_gather(table_v, [rowbase + d])
                    plsc.store_scatter(
                        pan, [zerov, jnp.full((_L,), d, jnp.int32), bv], vals
                    )

        def fire_store(h, b):
            pltpu.async_copy(
                panels[b],
                out_hbm.at[pl.ds(h, 1), :, pl.ds(bcol, b_per_w)],
                ssem[b],
            )

        def wait_store(b):
            pltpu.make_async_copy(
                out_hbm.at[pl.ds(0, 1), :, pl.ds(0, b_per_w)], panels[b], ssem[b]
            ).wait()

        def body(g, carry):
            for b in range(2):
                h = g * 2 + b

                @pl.when(g > 0)
                def _():
                    wait_store(b)

                fill(h, panels[b])
                fire_store(h, b)
            return carry

        lax.fori_loop(0, hist // 2, body, 0)
        for b in range(2):
            wait_store(b)

    return gather_kernel


def kernel(indices, table):
    batch, hist = indices.shape
    vocab, dim = table.shape
    idx_flat = indices.reshape(batch * hist)
    table_flat = table.reshape(vocab * dim)
    out = _make_gather(batch, hist, vocab, dim)(table_flat, idx_flat)
    return jnp.transpose(out, (2, 0, 1))


# speedup vs baseline: 13.2238x; 2.3191x over previous
import functools

import jax
import jax.numpy as jnp
from jax import lax
from jax.experimental import pallas as pl
from jax.experimental.pallas import tpu as pltpu
from jax.experimental.pallas import tpu_sc as plsc

_INFO = plsc.get_sparse_core_info()
_NC = _INFO.num_cores
_NS = _INFO.num_subcores
_NW = _NC * _NS
_L = _INFO.num_lanes


def _make_gather(batch, hist, vocab, dim):
    assert batch % (_NW * 128) == 0
    b_per_w = batch // _NW
    n_bblk = b_per_w // _L
    stride = dim + 1

    @functools.partial(
        pl.kernel,
        mesh=plsc.VectorSubcoreMesh(core_axis_name="c", subcore_axis_name="s"),
        out_type=jax.ShapeDtypeStruct((hist, dim, batch), jnp.float32),
        scratch_types=[
            pltpu.VMEM((b_per_w * hist,), jnp.int32),
            pltpu.VMEM((vocab * dim,), jnp.float32),
            pltpu.VMEM((vocab * stride,), jnp.float32),
            pltpu.VMEM((1, dim, b_per_w), jnp.float32),
            pltpu.VMEM((1, dim, b_per_w), jnp.float32),
            pltpu.SemaphoreType.DMA,
            pltpu.SemaphoreType.DMA,
        ],
        compiler_params=pltpu.CompilerParams(
            use_tc_tiling_on_sc=True, needs_layout_passes=False
        ),
    )
    def gather_kernel(
        table_hbm, idx_hbm, out_hbm, idx_v, stage_v, table_v, pan0, pan1, sem0, sem1
    ):
        panels = (pan0, pan1)
        ssem = (sem0, sem1)
        wid = lax.axis_index("s") * _NC + lax.axis_index("c")
        base = pl.multiple_of(wid * (b_per_w * hist), 8)
        pltpu.sync_copy(idx_hbm.at[pl.ds(base, b_per_w * hist)], idx_v)
        pltpu.sync_copy(table_hbm, stage_v)
        for r in range(vocab):
            for k in range(dim // _L):
                table_v[pl.ds(r * stride + k * _L, _L)] = stage_v[
                    pl.ds(r * dim + k * _L, _L)
                ]

        lane = lax.iota(jnp.int32, _L)
        bcol = pl.multiple_of(wid * b_per_w, 128)

        zerov = jnp.zeros((_L,), jnp.int32)
        ddc = [jnp.full((_L,), dd, jnp.int32) for dd in range(8)]

        def fill(h, pan):
            @plsc.parallel_loop(0, n_bblk)
            def blk_body(blk):
                b0 = blk * _L
                posv = jnp.full((_L,), b0 * hist + h, jnp.int32) + lane * hist
                idxv = plsc.load_gather(idx_v, [posv])
                rowbase = idxv * stride
                bv = jnp.full((_L,), b0, jnp.int32) + lane

                @plsc.parallel_loop(0, dim // 8)
                def d_body(dg):
                    d0v = jnp.full((_L,), dg * 8, jnp.int32)
                    rb = rowbase + d0v
                    for dd in range(8):
                        vals = plsc.load_gather(table_v, [rb + ddc[dd]])
                        plsc.store_scatter(
                            pan, [zerov, d0v + ddc[dd], bv], vals
                        )

        def fire_store(h, b):
            pltpu.async_copy(
                panels[b],
                out_hbm.at[pl.ds(h, 1), :, pl.ds(bcol, b_per_w)],
                ssem[b],
            )

        def wait_store(b):
            pltpu.make_async_copy(
                out_hbm.at[pl.ds(0, 1), :, pl.ds(0, b_per_w)], panels[b], ssem[b]
            ).wait()

        def body(g, carry):
            for b in range(2):
                h = g * 2 + b

                @pl.when(g > 0)
                def _():
                    wait_store(b)

                fill(h, panels[b])
                fire_store(h, b)
            return carry

        lax.fori_loop(0, hist // 2, body, 0)
        for b in range(2):
            wait_store(b)

    return gather_kernel


def kernel(indices, table):
    batch, hist = indices.shape
    vocab, dim = table.shape
    idx_flat = indices.reshape(batch * hist)
    table_flat = table.reshape(vocab * dim)
    out = _make_gather(batch, hist, vocab, dim)(table_flat, idx_flat)
    return jnp.transpose(out, (2, 0, 1))


# speedup vs baseline: 15.2622x; 1.1541x over previous
import functools

import jax
import jax.numpy as jnp
from jax import lax
from jax.experimental import pallas as pl
from jax.experimental.pallas import tpu as pltpu
from jax.experimental.pallas import tpu_sc as plsc

_INFO = plsc.get_sparse_core_info()
_NC = _INFO.num_cores
_NS = _INFO.num_subcores
_NW = _NC * _NS
_L = _INFO.num_lanes


def _make_gather(batch, hist, vocab, dim):
    assert batch % (_NW * 128) == 0
    b_per_w = batch // _NW
    n_bblk = b_per_w // _L
    stride = dim + 1

    @functools.partial(
        pl.kernel,
        mesh=plsc.VectorSubcoreMesh(core_axis_name="c", subcore_axis_name="s"),
        out_type=jax.ShapeDtypeStruct((hist, dim, batch), jnp.float32),
        scratch_types=[
            pltpu.VMEM((b_per_w * hist,), jnp.int32),
            pltpu.VMEM((vocab * dim,), jnp.float32),
            pltpu.VMEM((vocab * stride,), jnp.float32),
            pltpu.VMEM((1, dim, b_per_w), jnp.float32),
            pltpu.VMEM((1, dim, b_per_w), jnp.float32),
            pltpu.SemaphoreType.DMA,
            pltpu.SemaphoreType.DMA,
        ],
        compiler_params=pltpu.CompilerParams(
            use_tc_tiling_on_sc=True, needs_layout_passes=False
        ),
    )
    def gather_kernel(
        table_hbm, idx_hbm, out_hbm, idx_v, stage_v, table_v, pan0, pan1, sem0, sem1
    ):
        panels = (pan0, pan1)
        ssem = (sem0, sem1)
        wid = lax.axis_index("s") * _NC + lax.axis_index("c")
        base = pl.multiple_of(wid * (b_per_w * hist), 8)
        pltpu.sync_copy(idx_hbm.at[pl.ds(base, b_per_w * hist)], idx_v)
        pltpu.sync_copy(table_hbm, stage_v)
        for r in range(vocab):
            for k in range(dim // _L):
                table_v[pl.ds(r * stride + k * _L, _L)] = stage_v[
                    pl.ds(r * dim + k * _L, _L)
                ]

        lane = lax.iota(jnp.int32, _L)
        bcol = pl.multiple_of(wid * b_per_w, 128)

        zerov = jnp.zeros((_L,), jnp.int32)
        ddc = [jnp.full((_L,), dd, jnp.int32) for dd in range(8)]

        def fill(h, pan):
            @plsc.parallel_loop(0, n_bblk, unroll=2)
            def blk_body(blk):
                b0 = blk * _L
                posv = jnp.full((_L,), b0 * hist + h, jnp.int32) + lane * hist
                idxv = plsc.load_gather(idx_v, [posv])
                rowbase = idxv * stride
                bv = jnp.full((_L,), b0, jnp.int32) + lane

                @plsc.parallel_loop(0, dim // 8)
                def d_body(dg):
                    d0v = jnp.full((_L,), dg * 8, jnp.int32)
                    rb = rowbase + d0v
                    for dd in range(8):
                        vals = plsc.load_gather(table_v, [rb + ddc[dd]])
                        plsc.store_scatter(
                            pan, [zerov, d0v + ddc[dd], bv], vals
                        )

        def fire_store(h, b):
            pltpu.async_copy(
                panels[b],
                out_hbm.at[pl.ds(h, 1), :, pl.ds(bcol, b_per_w)],
                ssem[b],
            )

        def wait_store(b):
            pltpu.make_async_copy(
                out_hbm.at[pl.ds(0, 1), :, pl.ds(0, b_per_w)], panels[b], ssem[b]
            ).wait()

        def body(g, carry):
            for b in range(2):
                h = g * 2 + b

                @pl.when(g > 0)
                def _():
                    wait_store(b)

                fill(h, panels[b])
                fire_store(h, b)
            return carry

        lax.fori_loop(0, hist // 2, body, 0)
        for b in range(2):
            wait_store(b)

    return gather_kernel


def kernel(indices, table):
    batch, hist = indices.shape
    vocab, dim = table.shape
    idx_flat = indices.reshape(batch * hist)
    table_flat = table.reshape(vocab * dim)
    out = _make_gather(batch, hist, vocab, dim)(table_flat, idx_flat)
    return jnp.transpose(out, (2, 0, 1))


# speedup vs baseline: 16.5031x; 1.0813x over previous
import functools

import jax
import jax.numpy as jnp
from jax import lax
from jax.experimental import pallas as pl
from jax.experimental.pallas import tpu as pltpu
from jax.experimental.pallas import tpu_sc as plsc

_INFO = plsc.get_sparse_core_info()
_NC = _INFO.num_cores
_NS = _INFO.num_subcores
_NW = _NC * _NS
_L = _INFO.num_lanes


def _make_gather(batch, hist, vocab, dim):
    assert batch % (_NW * 128) == 0
    b_per_w = batch // _NW
    n_bblk = b_per_w // _L
    stride = dim + 1

    @functools.partial(
        pl.kernel,
        mesh=plsc.VectorSubcoreMesh(core_axis_name="c", subcore_axis_name="s"),
        out_type=jax.ShapeDtypeStruct((hist, dim, batch), jnp.float32),
        scratch_types=[
            pltpu.VMEM((b_per_w * hist,), jnp.int32),
            pltpu.VMEM((vocab * dim,), jnp.float32),
            pltpu.VMEM((vocab * stride,), jnp.float32),
            pltpu.VMEM((1, dim, b_per_w), jnp.float32),
            pltpu.VMEM((1, dim, b_per_w), jnp.float32),
            pltpu.SemaphoreType.DMA,
            pltpu.SemaphoreType.DMA,
        ],
        compiler_params=pltpu.CompilerParams(
            use_tc_tiling_on_sc=True, needs_layout_passes=False
        ),
    )
    def gather_kernel(
        table_hbm, idx_hbm, out_hbm, idx_v, stage_v, table_v, pan0, pan1, sem0, sem1
    ):
        panels = (pan0, pan1)
        ssem = (sem0, sem1)
        wid = lax.axis_index("s") * _NC + lax.axis_index("c")
        base = pl.multiple_of(wid * (b_per_w * hist), 8)
        pltpu.sync_copy(idx_hbm.at[pl.ds(base, b_per_w * hist)], idx_v)
        pltpu.sync_copy(table_hbm, stage_v)
        for r in range(vocab):
            for k in range(dim // _L):
                table_v[pl.ds(r * stride + k * _L, _L)] = stage_v[
                    pl.ds(r * dim + k * _L, _L)
                ]

        lane = lax.iota(jnp.int32, _L)
        bcol = pl.multiple_of(wid * b_per_w, 128)

        zerov = jnp.zeros((_L,), jnp.int32)
        ddc = [jnp.full((_L,), dd, jnp.int32) for dd in range(8)]

        def fill(h, pan):
            @plsc.parallel_loop(0, n_bblk, unroll=4)
            def blk_body(blk):
                b0 = blk * _L
                posv = jnp.full((_L,), b0 * hist + h, jnp.int32) + lane * hist
                idxv = plsc.load_gather(idx_v, [posv])
                rowbase = idxv * stride
                bv = jnp.full((_L,), b0, jnp.int32) + lane

                @plsc.parallel_loop(0, dim // 8)
                def d_body(dg):
                    d0v = jnp.full((_L,), dg * 8, jnp.int32)
                    rb = rowbase + d0v
                    for dd in range(8):
                        vals = plsc.load_gather(table_v, [rb + ddc[dd]])
                        plsc.store_scatter(
                            pan, [zerov, d0v + ddc[dd], bv], vals
                        )

        def fire_store(h, b):
            pltpu.async_copy(
                panels[b],
                out_hbm.at[pl.ds(h, 1), :, pl.ds(bcol, b_per_w)],
                ssem[b],
            )

        def wait_store(b):
            pltpu.make_async_copy(
                out_hbm.at[pl.ds(0, 1), :, pl.ds(0, b_per_w)], panels[b], ssem[b]
            ).wait()

        def body(g, carry):
            for b in range(2):
                h = g * 2 + b

                @pl.when(g > 0)
                def _():
                    wait_store(b)

                fill(h, panels[b])
                fire_store(h, b)
            return carry

        lax.fori_loop(0, hist // 2, body, 0)
        for b in range(2):
            wait_store(b)

    return gather_kernel


def kernel(indices, table):
    batch, hist = indices.shape
    vocab, dim = table.shape
    idx_flat = indices.reshape(batch * hist)
    table_flat = table.reshape(vocab * dim)
    out = _make_gather(batch, hist, vocab, dim)(table_flat, idx_flat)
    return jnp.transpose(out, (2, 0, 1))
